# baseline (speedup 1.0000x reference)
import functools

import jax
import jax.numpy as jnp
from jax import lax
from jax.experimental import pallas as pl
from jax.experimental.pallas import tpu as pltpu
from jax.experimental.pallas import tpu_sc as plsc

N_NODES = 10000
N_EDGES = 320000
D = 128

NC = 2
NS = 16
NW = NC * NS

EDGES_PER_TILE = N_EDGES // NW
CHUNK = 80
NCHUNK = EDGES_PER_TILE // CHUNK
DSTAGE = 64
N_PAD = 10112
ROWS_PER_TILE = N_PAD // NS

_mesh = plsc.VectorSubcoreMesh(core_axis_name="c", subcore_axis_name="s")


@functools.partial(
    pl.kernel,
    mesh=_mesh,
    out_type=jax.ShapeDtypeStruct((NC, N_PAD, D), jnp.float32),
    scratch_types=[
        pltpu.VMEM((EDGES_PER_TILE,), jnp.int32),
        pltpu.VMEM((DSTAGE, CHUNK), jnp.int32),
        pltpu.VMEM((3, CHUNK, D), jnp.float32),
        pltpu.VMEM_SHARED((N_PAD, D), jnp.float32),
        [pltpu.SemaphoreType.DMA] * 3,
        [pltpu.SemaphoreType.DMA] * 3,
        [pltpu.SemaphoreType.DMA] * 3,
    ],
)
def _sc_aggregate(feat_hbm, src_hbm, dst_hbm, zeros_hbm, part_hbm,
                  src_v, dst_v, rows_v, acc_sh, sg, sg2, ss):
    c = lax.axis_index("c")
    s = lax.axis_index("s")
    wid = c * NS + s

    pltpu.sync_copy(zeros_hbm.at[pl.ds(s * ROWS_PER_TILE, ROWS_PER_TILE)],
                    acc_sh.at[pl.ds(s * ROWS_PER_TILE, ROWS_PER_TILE)])

    pltpu.sync_copy(src_hbm.at[wid], src_v)
    pltpu.sync_copy(dst_hbm.at[wid].at[pl.ds(0, DSTAGE)], dst_v)
    plsc.subcore_barrier()

    H = CHUNK // 2

    def half(i, b, lo, sem):
        return pltpu.make_async_copy(
            feat_hbm.at[src_v.at[pl.ds(i * CHUNK + lo, H)]],
            rows_v.at[b].at[pl.ds(lo, H)], sem)

    def issue_g(i, b):
        half(i, b, 0, sg[b]).start()
        half(i, b, H, sg2[b]).start()

    def wait_g(i, b):
        half(i, b, 0, sg[b]).wait()
        half(i, b, H, sg2[b]).wait()

    def issue_s(row, b):
        pltpu.async_copy(rows_v.at[b], acc_sh.at[dst_v.at[row]], ss[b],
                         add=True)

    def wait_s(row, b):
        pltpu.make_async_copy(rows_v.at[b], acc_sh.at[dst_v.at[row]],
                              ss[b]).wait()

    def full(i, b, row):
        wait_g(i, b)
        issue_s(row, b)
        wait_s(row, b)
        issue_g(i + 3, b)

    def tail(i, b, row):
        wait_g(i, b)
        issue_s(row, b)
        wait_s(row, b)

    for b in range(3):
        issue_g(b, b)

    def body_a(it, _):
        for m in range(3):
            i = it * 3 + m
            full(i, m, i)
        return 0

    lax.fori_loop(0, (DSTAGE - 1) // 3, body_a, 0)
    full(DSTAGE - 1, (DSTAGE - 1) % 3, DSTAGE - 1)

    pltpu.sync_copy(dst_hbm.at[wid].at[pl.ds(DSTAGE, NCHUNK - DSTAGE)],
                    dst_v.at[pl.ds(0, NCHUNK - DSTAGE)])

    def body_b(it, _):
        for m in range(3):
            i = DSTAGE + it * 3 + m
            full(i, (DSTAGE + m) % 3, i - DSTAGE)
        return 0

    lax.fori_loop(0, 19, body_b, 0)
    full(NCHUNK - 4, (NCHUNK - 4) % 3, NCHUNK - 4 - DSTAGE)
    for i in range(NCHUNK - 3, NCHUNK):
        tail(i, i % 3, i - DSTAGE)

    plsc.subcore_barrier()
    pltpu.sync_copy(acc_sh.at[pl.ds(s * ROWS_PER_TILE, ROWS_PER_TILE)],
                    part_hbm.at[c].at[pl.ds(s * ROWS_PER_TILE, ROWS_PER_TILE)])


_ROW_BLK = 1000


def _tc_body(p_ref, wt_ref, b_ref, o_ref):
    agg = p_ref[0] + p_ref[1]
    o_ref[...] = (jnp.dot(agg, wt_ref[...], preferred_element_type=jnp.float32)
                  + b_ref[...])


def _tc_linear(partials, wt, b2):
    return pl.pallas_call(
        _tc_body,
        grid=(N_NODES // _ROW_BLK,),
        in_specs=[
            pl.BlockSpec((NC, _ROW_BLK, D), lambda i: (0, i, 0)),
            pl.BlockSpec((D, D), lambda i: (0, 0)),
            pl.BlockSpec((1, D), lambda i: (0, 0)),
        ],
        out_specs=pl.BlockSpec((_ROW_BLK, D), lambda i: (i, 0)),
        out_shape=jax.ShapeDtypeStruct((N_NODES, D), jnp.float32),
    )(partials, wt, b2)


def kernel(features, edge_index, W, b):
    src = edge_index[0].astype(jnp.int32).reshape(NW, EDGES_PER_TILE)
    dst = edge_index[1].astype(jnp.int32).reshape(NW, NCHUNK, CHUNK)
    zeros = jnp.zeros((N_PAD, D), jnp.float32)
    partials = _sc_aggregate(features, src, dst, zeros)
    return _tc_linear(partials, W.T, b.reshape(1, D))


# speedup vs baseline: 1.0362x; 1.0362x over previous
import functools

import jax
import jax.numpy as jnp
from jax import lax
from jax.experimental import pallas as pl
from jax.experimental.pallas import tpu as pltpu
from jax.experimental.pallas import tpu_sc as plsc

N_NODES = 10000
N_EDGES = 320000
D = 128

NC = 2
NS = 16
NW = NC * NS

EDGES_PER_TILE = N_EDGES // NW
CHUNK = 80
NCHUNK = EDGES_PER_TILE // CHUNK
DSTAGE = 64
N_PAD = 10112
ROWS_PER_TILE = N_PAD // NS

_mesh = plsc.VectorSubcoreMesh(core_axis_name="c", subcore_axis_name="s")


@functools.partial(
    pl.kernel,
    mesh=_mesh,
    out_type=jax.ShapeDtypeStruct((NC, N_PAD, D), jnp.float32),
    scratch_types=[
        pltpu.VMEM((EDGES_PER_TILE,), jnp.int32),
        pltpu.VMEM((DSTAGE, CHUNK), jnp.int32),
        pltpu.VMEM((3, CHUNK, D), jnp.float32),
        pltpu.VMEM_SHARED((N_PAD, D), jnp.float32),
        [pltpu.SemaphoreType.DMA] * 3,
        [pltpu.SemaphoreType.DMA] * 3,
        [pltpu.SemaphoreType.DMA] * 3,
    ],
)
def _sc_aggregate(feat_hbm, src_hbm, dst_hbm, part_hbm,
                  src_v, dst_v, rows_v, acc_sh, sg, sg2, ss):
    c = lax.axis_index("c")
    s = lax.axis_index("s")
    wid = c * NS + s

    pltpu.sync_copy(src_hbm.at[wid], src_v)
    pltpu.sync_copy(dst_hbm.at[wid].at[pl.ds(0, DSTAGE)], dst_v)

    def zrow(r, _):
        for j in range(D // 16):
            rows_v[0, r, pl.ds(j * 16, 16)] = jnp.zeros((16,), jnp.float32)
        return 0

    lax.fori_loop(0, CHUNK, zrow, 0)
    base = s * ROWS_PER_TILE
    for k in range(ROWS_PER_TILE // CHUNK):
        pltpu.sync_copy(rows_v.at[0],
                        acc_sh.at[pl.ds(base + k * CHUNK, CHUNK)])
    rem = ROWS_PER_TILE % CHUNK
    if rem:
        pltpu.sync_copy(
            rows_v.at[0].at[pl.ds(0, rem)],
            acc_sh.at[pl.ds(base + ROWS_PER_TILE - rem, rem)])
    plsc.subcore_barrier()

    H = CHUNK // 2

    def half(i, b, lo, sem):
        return pltpu.make_async_copy(
            feat_hbm.at[src_v.at[pl.ds(i * CHUNK + lo, H)]],
            rows_v.at[b].at[pl.ds(lo, H)], sem)

    def issue_g(i, b):
        half(i, b, 0, sg[b]).start()
        half(i, b, H, sg2[b]).start()

    def wait_g(i, b):
        half(i, b, 0, sg[b]).wait()
        half(i, b, H, sg2[b]).wait()

    def issue_s(row, b):
        pltpu.async_copy(rows_v.at[b], acc_sh.at[dst_v.at[row]], ss[b],
                         add=True)

    def wait_s(row, b):
        pltpu.make_async_copy(rows_v.at[b], acc_sh.at[dst_v.at[row]],
                              ss[b]).wait()

    def full(i, b, row):
        wait_g(i, b)
        issue_s(row, b)
        wait_s(row, b)
        issue_g(i + 3, b)

    def tail(i, b, row):
        wait_g(i, b)
        issue_s(row, b)
        wait_s(row, b)

    for b in range(3):
        issue_g(b, b)

    def body_a(it, _):
        for m in range(3):
            i = it * 3 + m
            full(i, m, i)
        return 0

    lax.fori_loop(0, (DSTAGE - 1) // 3, body_a, 0)
    full(DSTAGE - 1, (DSTAGE - 1) % 3, DSTAGE - 1)

    pltpu.sync_copy(dst_hbm.at[wid].at[pl.ds(DSTAGE, NCHUNK - DSTAGE)],
                    dst_v.at[pl.ds(0, NCHUNK - DSTAGE)])

    def body_b(it, _):
        for m in range(3):
            i = DSTAGE + it * 3 + m
            full(i, (DSTAGE + m) % 3, i - DSTAGE)
        return 0

    lax.fori_loop(0, 19, body_b, 0)
    full(NCHUNK - 4, (NCHUNK - 4) % 3, NCHUNK - 4 - DSTAGE)
    for i in range(NCHUNK - 3, NCHUNK):
        tail(i, i % 3, i - DSTAGE)

    plsc.subcore_barrier()
    pltpu.sync_copy(acc_sh.at[pl.ds(s * ROWS_PER_TILE, ROWS_PER_TILE)],
                    part_hbm.at[c].at[pl.ds(s * ROWS_PER_TILE, ROWS_PER_TILE)])


_ROW_BLK = 1000


def _tc_body(p_ref, wt_ref, b_ref, o_ref):
    agg = p_ref[0] + p_ref[1]
    o_ref[...] = (jnp.dot(agg, wt_ref[...], preferred_element_type=jnp.float32)
                  + b_ref[...])


def _tc_linear(partials, wt, b2):
    return pl.pallas_call(
        _tc_body,
        grid=(N_NODES // _ROW_BLK,),
        in_specs=[
            pl.BlockSpec((NC, _ROW_BLK, D), lambda i: (0, i, 0)),
            pl.BlockSpec((D, D), lambda i: (0, 0)),
            pl.BlockSpec((1, D), lambda i: (0, 0)),
        ],
        out_specs=pl.BlockSpec((_ROW_BLK, D), lambda i: (i, 0)),
        out_shape=jax.ShapeDtypeStruct((N_NODES, D), jnp.float32),
    )(partials, wt, b2)


def kernel(features, edge_index, W, b):
    src = edge_index[0].astype(jnp.int32).reshape(NW, EDGES_PER_TILE)
    dst = edge_index[1].astype(jnp.int32).reshape(NW, NCHUNK, CHUNK)
    partials = _sc_aggregate(features, src, dst)
    return _tc_linear(partials, W.T, b.reshape(1, D))


# speedup vs baseline: 1.1002x; 1.0617x over previous
import functools

import jax
import jax.numpy as jnp
from jax import lax
from jax.experimental import pallas as pl
from jax.experimental.pallas import tpu as pltpu
from jax.experimental.pallas import tpu_sc as plsc

N_NODES = 10000
N_EDGES = 320000
D = 128

NC = 2
NS = 16
NW = NC * NS

EDGES_PER_TILE = N_EDGES // NW
CHUNK = 80
NCHUNK = EDGES_PER_TILE // CHUNK
HSTAGE = 64
N_PAD = 10112
ROWS_PER_TILE = N_PAD // NS

_mesh = plsc.VectorSubcoreMesh(core_axis_name="c", subcore_axis_name="s")


@functools.partial(
    pl.kernel,
    mesh=_mesh,
    out_type=jax.ShapeDtypeStruct((NC, N_PAD, D), jnp.float32),
    scratch_types=[
        pltpu.VMEM((HSTAGE, CHUNK), jnp.int32),
        pltpu.VMEM((HSTAGE, CHUNK), jnp.int32),
        pltpu.VMEM((3, CHUNK, D), jnp.float32),
        pltpu.VMEM_SHARED((N_PAD, D), jnp.float32),
        [pltpu.SemaphoreType.DMA] * 3,
        [pltpu.SemaphoreType.DMA] * 3,
    ],
)
def _sc_aggregate(feat_hbm, edge_hbm, part_hbm,
                  src_v, dst_v, rows_v, acc_sh, sg, ss):
    c = lax.axis_index("c")
    s = lax.axis_index("s")
    wid = c * NS + s
    src_t = edge_hbm.at[0].at[wid]
    dst_t = edge_hbm.at[1].at[wid]

    pltpu.sync_copy(src_t.at[pl.ds(0, HSTAGE)], src_v)
    pltpu.sync_copy(dst_t.at[pl.ds(0, HSTAGE)], dst_v)

    def zrow(r, _):
        for j in range(D // 16):
            rows_v[0, r, pl.ds(j * 16, 16)] = jnp.zeros((16,), jnp.float32)
        return 0

    lax.fori_loop(0, CHUNK, zrow, 0)
    base = s * ROWS_PER_TILE
    for k in range(ROWS_PER_TILE // CHUNK):
        pltpu.sync_copy(rows_v.at[0],
                        acc_sh.at[pl.ds(base + k * CHUNK, CHUNK)])
    rem = ROWS_PER_TILE % CHUNK
    if rem:
        pltpu.sync_copy(
            rows_v.at[0].at[pl.ds(0, rem)],
            acc_sh.at[pl.ds(base + ROWS_PER_TILE - rem, rem)])
    plsc.subcore_barrier()

    def issue_g(r, b):
        pltpu.async_copy(feat_hbm.at[src_v.at[r]], rows_v.at[b], sg[b])

    def wait_g(r, b):
        pltpu.make_async_copy(feat_hbm.at[src_v.at[r]], rows_v.at[b],
                              sg[b]).wait()

    def issue_s(r, b):
        pltpu.async_copy(rows_v.at[b], acc_sh.at[dst_v.at[r]], ss[b],
                         add=True)

    def wait_s(r, b):
        pltpu.make_async_copy(rows_v.at[b], acc_sh.at[dst_v.at[r]],
                              ss[b]).wait()

    def full(r, b):
        wait_g(r, b)
        issue_s(r, b)
        wait_s(r, b)
        issue_g(r + 3, b)

    def tail(r, b):
        wait_g(r, b)
        issue_s(r, b)
        wait_s(r, b)

    for b in range(3):
        issue_g(b, b)

    def body_a(it, _):
        for m in range(3):
            full(it * 3 + m, m)
        return 0

    lax.fori_loop(0, (HSTAGE - 4) // 3, body_a, 0)
    full(HSTAGE - 4, (HSTAGE - 4) % 3)
    for r in range(HSTAGE - 3, HSTAGE):
        tail(r, r % 3)

    half_b = NCHUNK - HSTAGE
    pltpu.sync_copy(src_t.at[pl.ds(HSTAGE, half_b)],
                    src_v.at[pl.ds(0, half_b)])
    pltpu.sync_copy(dst_t.at[pl.ds(HSTAGE, half_b)],
                    dst_v.at[pl.ds(0, half_b)])

    for r in range(3):
        issue_g(r, (HSTAGE + r) % 3)

    def body_b(it, _):
        for m in range(3):
            full(it * 3 + m, (HSTAGE + m) % 3)
        return 0

    lax.fori_loop(0, (half_b - 4) // 3, body_b, 0)
    full(half_b - 4, (HSTAGE + half_b - 4) % 3)
    for r in range(half_b - 3, half_b):
        tail(r, (HSTAGE + r) % 3)

    plsc.subcore_barrier()
    pltpu.sync_copy(acc_sh.at[pl.ds(s * ROWS_PER_TILE, ROWS_PER_TILE)],
                    part_hbm.at[c].at[pl.ds(s * ROWS_PER_TILE, ROWS_PER_TILE)])


_ROW_BLK = 1000


def _tc_body(p_ref, wt_ref, b_ref, o_ref):
    agg = p_ref[0] + p_ref[1]
    o_ref[...] = (jnp.dot(agg, wt_ref[...], preferred_element_type=jnp.float32)
                  + b_ref[...])


def _tc_linear(partials, wt, b2):
    return pl.pallas_call(
        _tc_body,
        grid=(N_NODES // _ROW_BLK,),
        in_specs=[
            pl.BlockSpec((NC, _ROW_BLK, D), lambda i: (0, i, 0)),
            pl.BlockSpec((D, D), lambda i: (0, 0)),
            pl.BlockSpec((1, D), lambda i: (0, 0)),
        ],
        out_specs=pl.BlockSpec((_ROW_BLK, D), lambda i: (i, 0)),
        out_shape=jax.ShapeDtypeStruct((N_NODES, D), jnp.float32),
    )(partials, wt, b2)


def kernel(features, edge_index, W, b):
    edges = edge_index.astype(jnp.int32).reshape(2, NW, NCHUNK, CHUNK)
    partials = _sc_aggregate(features, edges)
    return _tc_linear(partials, W.T, b.reshape(1, D))


# speedup vs baseline: 1.1211x; 1.0190x over previous
import functools

import jax
import jax.numpy as jnp
from jax import lax
from jax.experimental import pallas as pl
from jax.experimental.pallas import tpu as pltpu
from jax.experimental.pallas import tpu_sc as plsc

N_NODES = 10000
N_EDGES = 320000
D = 128

NC = 2
NS = 16
NW = NC * NS

EDGES_PER_TILE = N_EDGES // NW
CHUNK = 80
NCHUNK = EDGES_PER_TILE // CHUNK
HSTAGE = 64
N_PAD = 10112
ROWS_PER_TILE = N_PAD // NS

_mesh = plsc.VectorSubcoreMesh(core_axis_name="c", subcore_axis_name="s")


@functools.partial(
    pl.kernel,
    mesh=_mesh,
    out_type=jax.ShapeDtypeStruct((NC, N_PAD, D), jnp.float32),
    scratch_types=[
        pltpu.VMEM((HSTAGE, CHUNK), jnp.int32),
        pltpu.VMEM((HSTAGE, CHUNK), jnp.int32),
        pltpu.VMEM((3, CHUNK, D), jnp.float32),
        pltpu.VMEM_SHARED((N_PAD, D), jnp.float32),
        [pltpu.SemaphoreType.DMA] * 3,
        [pltpu.SemaphoreType.DMA] * 3,
    ],
)
def _sc_aggregate(feat_hbm, edge_hbm, part_hbm,
                  src_v, dst_v, rows_v, acc_sh, sg, ss):
    c = lax.axis_index("c")
    s = lax.axis_index("s")
    wid = c * NS + s
    src_t = edge_hbm.at[0].at[wid]
    dst_t = edge_hbm.at[1].at[wid]

    pltpu.sync_copy(src_t.at[pl.ds(0, HSTAGE)], src_v)
    pltpu.sync_copy(dst_t.at[pl.ds(0, HSTAGE)], dst_v)

    def zrow(r, _):
        for j in range(D // 16):
            rows_v[0, r, pl.ds(j * 16, 16)] = jnp.zeros((16,), jnp.float32)
        return 0

    lax.fori_loop(0, CHUNK, zrow, 0)
    base = s * ROWS_PER_TILE
    for k in range(ROWS_PER_TILE // CHUNK):
        pltpu.sync_copy(rows_v.at[0],
                        acc_sh.at[pl.ds(base + k * CHUNK, CHUNK)])
    rem = ROWS_PER_TILE % CHUNK
    if rem:
        pltpu.sync_copy(
            rows_v.at[0].at[pl.ds(0, rem)],
            acc_sh.at[pl.ds(base + ROWS_PER_TILE - rem, rem)])
    plsc.subcore_barrier()

    def issue_g(r, b):
        pltpu.async_copy(feat_hbm.at[src_v.at[r]], rows_v.at[b], sg[b])

    def wait_g(r, b):
        pltpu.make_async_copy(feat_hbm.at[src_v.at[r]], rows_v.at[b],
                              sg[b]).wait()

    def issue_s(r, b):
        pltpu.async_copy(rows_v.at[b], acc_sh.at[dst_v.at[r]], ss[b],
                         add=True)

    def wait_s(r, b):
        pltpu.make_async_copy(rows_v.at[b], acc_sh.at[dst_v.at[r]],
                              ss[b]).wait()

    def full(r, b):
        wait_g(r, b)
        issue_s(r, b)
        wait_s(r, b)
        issue_g(r + 3, b)

    def tail(r, b):
        wait_g(r, b)
        issue_s(r, b)
        wait_s(r, b)

    for b in range(3):
        issue_g(b, b)

    def body_a(it, _):
        for m in range(3):
            full(it * 3 + m, m)
        return 0

    lax.fori_loop(0, (HSTAGE - 4) // 3, body_a, 0)
    full(HSTAGE - 4, (HSTAGE - 4) % 3)
    for r in range(HSTAGE - 3, HSTAGE):
        tail(r, r % 3)

    half_b = NCHUNK - HSTAGE
    pltpu.sync_copy(src_t.at[pl.ds(HSTAGE, half_b)],
                    src_v.at[pl.ds(0, half_b)])
    pltpu.sync_copy(dst_t.at[pl.ds(HSTAGE, half_b)],
                    dst_v.at[pl.ds(0, half_b)])

    for r in range(3):
        issue_g(r, (HSTAGE + r) % 3)

    def body_b(it, _):
        for m in range(3):
            full(it * 3 + m, (HSTAGE + m) % 3)
        return 0

    lax.fori_loop(0, (half_b - 4) // 3, body_b, 0)
    full(half_b - 4, (HSTAGE + half_b - 4) % 3)
    for r in range(half_b - 3, half_b):
        tail(r, (HSTAGE + r) % 3)

    plsc.subcore_barrier()
    pltpu.sync_copy(acc_sh.at[pl.ds(s * ROWS_PER_TILE, ROWS_PER_TILE)],
                    part_hbm.at[c].at[pl.ds(s * ROWS_PER_TILE, ROWS_PER_TILE)])


_ROW_BLK = 2000


def _tc_body(p_ref, wt_ref, b_ref, o_ref):
    agg = p_ref[0] + p_ref[1]
    o_ref[...] = (jnp.dot(agg, wt_ref[...], preferred_element_type=jnp.float32)
                  + b_ref[...])


def _tc_linear(partials, wt, b2):
    return pl.pallas_call(
        _tc_body,
        grid=(N_NODES // _ROW_BLK,),
        in_specs=[
            pl.BlockSpec((NC, _ROW_BLK, D), lambda i: (0, i, 0)),
            pl.BlockSpec((D, D), lambda i: (0, 0)),
            pl.BlockSpec((1, D), lambda i: (0, 0)),
        ],
        out_specs=pl.BlockSpec((_ROW_BLK, D), lambda i: (i, 0)),
        out_shape=jax.ShapeDtypeStruct((N_NODES, D), jnp.float32),
    )(partials, wt, b2)


def kernel(features, edge_index, W, b):
    edges = edge_index.astype(jnp.int32).reshape(2, NW, NCHUNK, CHUNK)
    partials = _sc_aggregate(features, edges)
    return _tc_linear(partials, W.T, b.reshape(1, D))


# speedup vs baseline: 1.1391x; 1.0161x over previous
import functools

import jax
import jax.numpy as jnp
from jax import lax
from jax.experimental import pallas as pl
from jax.experimental.pallas import tpu as pltpu
from jax.experimental.pallas import tpu_sc as plsc

N_NODES = 10000
N_EDGES = 320000
D = 128

NC = 2
NS = 16
NW = NC * NS

EDGES_PER_TILE = N_EDGES // NW
CHUNK = 80
NCHUNK = EDGES_PER_TILE // CHUNK
HSTAGE = 64
N_PAD = 10112
ROWS_PER_TILE = N_PAD // NS

_mesh = plsc.VectorSubcoreMesh(core_axis_name="c", subcore_axis_name="s")


@functools.partial(
    pl.kernel,
    mesh=_mesh,
    out_type=jax.ShapeDtypeStruct((NC, N_PAD, D), jnp.float32),
    scratch_types=[
        pltpu.VMEM((HSTAGE, CHUNK), jnp.int32),
        pltpu.VMEM((HSTAGE, CHUNK), jnp.int32),
        pltpu.VMEM((3, CHUNK, D), jnp.float32),
        pltpu.VMEM_SHARED((N_PAD, D), jnp.float32),
        [pltpu.SemaphoreType.DMA] * 3,
        [pltpu.SemaphoreType.DMA] * 3,
    ],
)
def _sc_aggregate(feat_hbm, edge_hbm, part_hbm,
                  src_v, dst_v, rows_v, acc_sh, sg, ss):
    c = lax.axis_index("c")
    s = lax.axis_index("s")
    wid = c * NS + s
    src_t = edge_hbm.at[0].at[wid]
    dst_t = edge_hbm.at[1].at[wid]

    pltpu.sync_copy(src_t.at[pl.ds(0, HSTAGE)], src_v)
    pltpu.sync_copy(dst_t.at[pl.ds(0, HSTAGE)], dst_v)

    def zrow(r, _):
        for j in range(D // 16):
            rows_v[0, r, pl.ds(j * 16, 16)] = jnp.zeros((16,), jnp.float32)
        return 0

    lax.fori_loop(0, CHUNK, zrow, 0)
    base = s * ROWS_PER_TILE
    for k in range(ROWS_PER_TILE // CHUNK):
        pltpu.sync_copy(rows_v.at[0],
                        acc_sh.at[pl.ds(base + k * CHUNK, CHUNK)])
    rem = ROWS_PER_TILE % CHUNK
    if rem:
        pltpu.sync_copy(
            rows_v.at[0].at[pl.ds(0, rem)],
            acc_sh.at[pl.ds(base + ROWS_PER_TILE - rem, rem)])
    plsc.subcore_barrier()

    def issue_g(r, b):
        pltpu.async_copy(feat_hbm.at[src_v.at[r]], rows_v.at[b], sg[b])

    def wait_g(r, b):
        pltpu.make_async_copy(feat_hbm.at[src_v.at[r]], rows_v.at[b],
                              sg[b]).wait()

    def issue_s(r, b):
        pltpu.async_copy(rows_v.at[b], acc_sh.at[dst_v.at[r]], ss[b],
                         add=True)

    def wait_s(r, b):
        pltpu.make_async_copy(rows_v.at[b], acc_sh.at[dst_v.at[r]],
                              ss[b]).wait()

    def full(r, b):
        wait_g(r, b)
        issue_s(r, b)
        wait_s(r, b)
        issue_g(r + 3, b)

    def tail(r, b):
        wait_g(r, b)
        issue_s(r, b)
        wait_s(r, b)

    for b in range(3):
        issue_g(b, b)

    def body_a(it, _):
        for m in range(3):
            full(it * 3 + m, m)
        return 0

    lax.fori_loop(0, (HSTAGE - 4) // 3, body_a, 0)
    full(HSTAGE - 4, (HSTAGE - 4) % 3)
    for r in range(HSTAGE - 3, HSTAGE):
        tail(r, r % 3)

    half_b = NCHUNK - HSTAGE
    pltpu.sync_copy(src_t.at[pl.ds(HSTAGE, half_b)],
                    src_v.at[pl.ds(0, half_b)])
    pltpu.sync_copy(dst_t.at[pl.ds(HSTAGE, half_b)],
                    dst_v.at[pl.ds(0, half_b)])

    for r in range(3):
        issue_g(r, (HSTAGE + r) % 3)

    def body_b(it, _):
        for m in range(3):
            full(it * 3 + m, (HSTAGE + m) % 3)
        return 0

    lax.fori_loop(0, (half_b - 4) // 3, body_b, 0)
    full(half_b - 4, (HSTAGE + half_b - 4) % 3)
    for r in range(half_b - 3, half_b):
        tail(r, (HSTAGE + r) % 3)

    plsc.subcore_barrier()
    pltpu.sync_copy(acc_sh.at[pl.ds(s * ROWS_PER_TILE, ROWS_PER_TILE)],
                    part_hbm.at[c].at[pl.ds(s * ROWS_PER_TILE, ROWS_PER_TILE)])


_ROW_BLK = 5000


def _tc_body(p_ref, wt_ref, b_ref, o_ref):
    agg = p_ref[0] + p_ref[1]
    o_ref[...] = (jnp.dot(agg, wt_ref[...], preferred_element_type=jnp.float32)
                  + b_ref[...])


def _tc_linear(partials, wt, b2):
    return pl.pallas_call(
        _tc_body,
        grid=(N_NODES // _ROW_BLK,),
        in_specs=[
            pl.BlockSpec((NC, _ROW_BLK, D), lambda i: (0, i, 0)),
            pl.BlockSpec((D, D), lambda i: (0, 0)),
            pl.BlockSpec((1, D), lambda i: (0, 0)),
        ],
        out_specs=pl.BlockSpec((_ROW_BLK, D), lambda i: (i, 0)),
        out_shape=jax.ShapeDtypeStruct((N_NODES, D), jnp.float32),
    )(partials, wt, b2)


def kernel(features, edge_index, W, b):
    edges = edge_index.astype(jnp.int32).reshape(2, NW, NCHUNK, CHUNK)
    partials = _sc_aggregate(features, edges)
    return _tc_linear(partials, W.T, b.reshape(1, D))
